# baseline (speedup 1.0000x reference)
import jax
import jax.numpy as jnp
from jax import lax
from jax.experimental import pallas as pl
from jax.experimental.pallas import tpu as pltpu
from jax.experimental.pallas import tpu_sc as plsc

N_NODES = 10000
N_GRAPHS = 16
NODES_PER = 625
E_EDGES = 320000
D_FEAT = 128
HIDDEN = 16
LATENT = 8
N_LAYERS = 10

ADJ_WORDS = N_GRAPHS * NODES_PER * NODES_PER

N_TASKS = 64
CHUNK = 97656
LAST_CHUNK = ADJ_WORDS - 63 * CHUNK
ACC_WORDS = 97680
EDGE_BATCH = 8000
N_BATCHES = E_EDGES // EDGE_BATCH


def _sc_scatter_body(idx_hbm, out_hbm, acc, ebuf):
    core = lax.axis_index("c")
    sub = lax.axis_index("s")
    wid = sub * 2 + core

    for p in range(2):
        task = wid + 32 * p
        base = task * CHUNK
        is_last = task == N_TASKS - 1
        end = base + jnp.where(is_last, LAST_CHUNK, CHUNK)

        zeros16 = jnp.zeros((16,), jnp.float32)

        def _zero(i, _):
            acc[pl.ds(i * 16, 16)] = zeros16
            return 0

        lax.fori_loop(0, ACC_WORDS // 16, _zero, 0, unroll=15)

        def _batch(b, _):
            pltpu.sync_copy(idx_hbm.at[pl.ds(b * EDGE_BATCH, EDGE_BATCH)], ebuf)

            def _vec(j, _):
                v = ebuf[pl.ds(j * 16, 16)]
                inb = (v >= base) & (v < end)
                li = jnp.where(inb, v - base, 0)
                val = jnp.where(inb, 1.0, 0.0).astype(jnp.float32)
                plsc.addupdate_scatter(acc, [li], val)
                return 0

            lax.fori_loop(0, EDGE_BATCH // 16, _vec, 0, unroll=10)
            return 0

        lax.fori_loop(0, N_BATCHES, _batch, 0)

        @pl.when(is_last)
        def _():
            pltpu.sync_copy(acc.at[pl.ds(0, LAST_CHUNK)],
                            out_hbm.at[pl.ds(base, LAST_CHUNK)])

        @pl.when(jnp.logical_not(is_last))
        def _():
            pltpu.sync_copy(acc.at[pl.ds(0, CHUNK)],
                            out_hbm.at[pl.ds(base, CHUNK)])


def _build_dense_adj(flat_idx):
    mesh = plsc.VectorSubcoreMesh(core_axis_name="c", subcore_axis_name="s")
    fn = pl.kernel(
        _sc_scatter_body,
        out_type=jax.ShapeDtypeStruct((ADJ_WORDS,), jnp.float32),
        mesh=mesh,
        scratch_types=[
            pltpu.VMEM((ACC_WORDS,), jnp.float32),
            pltpu.VMEM((EDGE_BATCH,), jnp.int32),
        ],
        compiler_params=pltpu.CompilerParams(needs_layout_passes=False),
    )
    return fn(flat_idx)


CAP = 12000
MCH = 2000
TRASH = 320


def _agg_body_factory(feat):
    nv = feat // 16

    def body(h_hbm, srcs_hbm, flags_hbm, dstl_hbm, out_hbm,
             sbuf, fbuf, dbuf, htile, obuf):
        core = lax.axis_index("c")
        sub = lax.axis_index("s")
        wid = sub * 2 + core
        g = wid // 2

        pltpu.sync_copy(h_hbm.at[g], htile)

        acc = tuple(jnp.zeros((16,), jnp.float32) for _ in range(nv))
        for c in range(CAP // MCH):
            moff = wid * CAP + c * MCH
            pltpu.sync_copy(srcs_hbm.at[pl.ds(moff, MCH)], sbuf)
            pltpu.sync_copy(flags_hbm.at[pl.ds(moff, MCH)], fbuf)
            pltpu.sync_copy(dstl_hbm.at[pl.ds(moff, MCH)], dbuf)

            def _edge16(jj, carry):
                sv = sbuf[pl.ds(jj * 16, 16)]
                fv = fbuf[pl.ds(jj * 16, 16)]
                dv = dbuf[pl.ds(jj * 16, 16)]
                a = carry
                for k in range(16):
                    si = sv[k]
                    ff = fv[k]
                    dl = dv[k]
                    new = []
                    for v in range(nv):
                        row = htile[si, pl.ds(v * 16, 16)]
                        av = a[v] * ff + row
                        obuf[dl, pl.ds(v * 16, 16)] = av
                        new.append(av)
                    a = tuple(new)
                return a

            acc = lax.fori_loop(0, MCH // 16, _edge16, acc)

        @pl.when(wid % 2 == 0)
        def _():
            pltpu.sync_copy(obuf.at[pl.ds(0, 320)],
                            out_hbm.at[g, pl.ds(0, 320)])

        @pl.when(wid % 2 == 1)
        def _():
            pltpu.sync_copy(obuf.at[pl.ds(0, 305)],
                            out_hbm.at[g, pl.ds(320, 305)])

    return body


def _make_agg(feat):
    mesh = plsc.VectorSubcoreMesh(core_axis_name="c", subcore_axis_name="s")
    return pl.kernel(
        _agg_body_factory(feat),
        out_type=jax.ShapeDtypeStruct((N_GRAPHS, NODES_PER, feat),
                                      jnp.float32),
        mesh=mesh,
        scratch_types=[
            pltpu.VMEM((MCH,), jnp.int32),
            pltpu.VMEM((MCH,), jnp.float32),
            pltpu.VMEM((MCH,), jnp.int32),
            pltpu.VMEM((NODES_PER, feat), jnp.float32),
            pltpu.VMEM((321, feat), jnp.float32),
        ],
        compiler_params=pltpu.CompilerParams(needs_layout_passes=False),
    )


def _mlp_body(last):
    def body(h_ref, agg_ref, w1_ref, b1_ref, w2_ref, b2_ref, out_ref):
        m = h_ref[...] + agg_ref[...]
        t = jnp.dot(m.astype(jnp.bfloat16), w1_ref[...],
                    preferred_element_type=jnp.float32) + b1_ref[...]
        t = jnp.maximum(t, 0.0)
        h2 = jnp.dot(t.astype(jnp.bfloat16), w2_ref[...],
                     preferred_element_type=jnp.float32) + b2_ref[...]
        if not last:
            h2 = jnp.maximum(h2, 0.0)
        out_ref[...] = h2
    return body


def _run_mlp(h, agg, w1b, b1, w2b, b2, last):
    full = lambda a: pl.BlockSpec(a.shape, lambda: (0,) * a.ndim)
    return pl.pallas_call(
        _mlp_body(last),
        in_specs=[full(h), full(agg), full(w1b), full(b1), full(w2b), full(b2)],
        out_specs=pl.BlockSpec((N_NODES, HIDDEN), lambda: (0, 0)),
        out_shape=jax.ShapeDtypeStruct((N_NODES, HIDDEN), jnp.float32),
    )(h, agg, w1b, b1, w2b, b2)


def _dec_body(h_ref, adj_ref, eps_ref, wm_ref, bm_ref, ws_ref, bs_ref,
              pred_ref, nll_ref, kl_ref):
    g = pl.program_id(0)
    hb = h_ref[0].astype(jnp.bfloat16)
    mu = jnp.dot(hb, wm_ref[...], preferred_element_type=jnp.float32) \
        + bm_ref[...]
    sx = jnp.dot(hb, ws_ref[...], preferred_element_type=jnp.float32) \
        + bs_ref[...]
    std = jnp.maximum(sx, 0.0) + jnp.log1p(jnp.exp(-jnp.abs(sx)))
    z = mu + std * eps_ref[0]
    zb = z.astype(jnp.bfloat16)
    logits = lax.dot_general(zb, zb, (((1,), (1,)), ((), ())),
                             preferred_element_type=jnp.float32)
    p = jax.nn.sigmoid(logits)
    pred_ref[0] = p

    adj = adj_ref[0]
    pc = jnp.clip(p, 1e-7, 1.0 - 1e-7)
    nll = -jnp.sum(adj * jnp.log(pc) + (1.0 - adj) * jnp.log(1.0 - pc))
    kl = jnp.sum(-jnp.log(std) + 0.5 * (std * std + mu * mu) - 0.5)

    @pl.when(g == 0)
    def _():
        nll_ref[...] = jnp.zeros((1, 1), jnp.float32)
        kl_ref[...] = jnp.zeros((1, 1), jnp.float32)

    nll_ref[...] += jnp.reshape(nll, (1, 1))
    kl_ref[...] += jnp.reshape(kl, (1, 1))


def _run_dec(h3, adj3, eps3, wmb, bm, wsb, bs):
    full = lambda a: pl.BlockSpec(a.shape, lambda g: (0,) * a.ndim)
    return pl.pallas_call(
        _dec_body,
        grid=(N_GRAPHS,),
        in_specs=[
            pl.BlockSpec((1, NODES_PER, HIDDEN), lambda g: (g, 0, 0)),
            pl.BlockSpec((1, NODES_PER, NODES_PER), lambda g: (g, 0, 0)),
            pl.BlockSpec((1, NODES_PER, LATENT), lambda g: (g, 0, 0)),
            full(wmb), full(bm), full(wsb), full(bs),
        ],
        out_specs=[
            pl.BlockSpec((1, NODES_PER, NODES_PER), lambda g: (g, 0, 0)),
            pl.BlockSpec((1, 1), lambda g: (0, 0)),
            pl.BlockSpec((1, 1), lambda g: (0, 0)),
        ],
        out_shape=[
            jax.ShapeDtypeStruct((N_GRAPHS, NODES_PER, NODES_PER), jnp.float32),
            jax.ShapeDtypeStruct((1, 1), jnp.float32),
            jax.ShapeDtypeStruct((1, 1), jnp.float32),
        ],
        compiler_params=pltpu.CompilerParams(
            dimension_semantics=("arbitrary",)),
    )(h3, adj3, eps3, wmb, bm, wsb, bs)


def kernel(x, edge_index, batch, params):
    src = edge_index[0].astype(jnp.int32)
    dst = edge_index[1].astype(jnp.int32)
    gsrc = src // NODES_PER
    flat_idx = (gsrc * NODES_PER + (src - gsrc * NODES_PER)) * NODES_PER \
        + (dst % NODES_PER)

    dense_flat = _build_dense_adj(flat_idx)
    dense_adj = dense_flat.reshape(N_GRAPHS, NODES_PER, NODES_PER)

    order = jnp.argsort(dst, stable=True)
    ssrc = src[order]
    sdst = dst[order]
    counts = jnp.zeros((N_NODES,), jnp.int32).at[sdst].add(1)
    starts_ext = jnp.concatenate(
        [jnp.zeros((1,), jnp.int32), jnp.cumsum(counts).astype(jnp.int32)])
    t = jnp.arange(32, dtype=jnp.int32)
    gt = t // 2
    d0 = gt * NODES_PER + (t % 2) * 320
    nd = jnp.where(t % 2 == 0, 320, 305)
    e0 = starts_ext[d0]
    e1 = starts_ext[d0 + nd]
    p = e0[:, None] + jnp.arange(CAP, dtype=jnp.int32)[None, :]
    valid = p < e1[:, None]
    pe = jnp.clip(p, 0, E_EDGES - 1)
    sd = sdst[pe]
    is_first = pe == starts_ext[sd]
    srcs_t = jnp.where(valid, ssrc[pe] - gt[:, None] * NODES_PER, 0).reshape(-1)
    flags_t = jnp.where(valid & ~is_first, 1.0, 0.0).astype(jnp.float32).reshape(-1)
    dstl_t = jnp.where(valid, sd - d0[:, None], TRASH).reshape(-1)

    agg128 = _make_agg(D_FEAT)
    agg16 = _make_agg(HIDDEN)

    bf = jnp.bfloat16
    eps = jax.random.normal(jax.random.key(42), (N_NODES, LATENT),
                            dtype=jnp.float32)

    h = x
    for i in range(N_LAYERS):
        pr = params['gin%d' % i]
        feat = D_FEAT if i == 0 else HIDDEN
        h3 = h.reshape(N_GRAPHS, NODES_PER, feat)
        agg = (agg128 if i == 0 else agg16)(h3, srcs_t, flags_t, dstl_t)
        agg = agg.reshape(N_NODES, feat)
        h = _run_mlp(h, agg,
                     pr['W1'].astype(bf), pr['b1'].reshape(1, HIDDEN),
                     pr['W2'].astype(bf), pr['b2'].reshape(1, HIDDEN),
                     last=(i == N_LAYERS - 1))

    h3 = h.reshape(N_GRAPHS, NODES_PER, HIDDEN)
    eps3 = eps.reshape(N_GRAPHS, NODES_PER, LATENT)
    adj_pred, nll, kl = _run_dec(
        h3, dense_adj, eps3,
        params['Wm'].astype(bf), params['bm'].reshape(1, LATENT),
        params['Ws'].astype(bf), params['bs'].reshape(1, LATENT))

    return nll[0, 0], kl[0, 0], adj_pred, dense_adj
